# baseline (speedup 1.0000x reference)
import functools

import jax
import jax.numpy as jnp
from jax import lax
from jax.experimental import pallas as pl
from jax.experimental.pallas import tpu as pltpu
from jax.experimental.pallas import tpu_sc as plsc

_B = 80

_SQRT1_2 = 0.7071067811865476
_P = 0.47047
_A1, _A2, _A3 = 0.3480242, -0.0958798, 0.7478556


def _rcp(d):
    i = lax.bitcast_convert_type(d, jnp.int32)
    r = lax.bitcast_convert_type(jnp.int32(0x7EF311C3) - i, jnp.float32)
    r = r * (2.0 - d * r)
    return r


def _rsqrt(d):
    i = lax.bitcast_convert_type(d, jnp.int32)
    y = lax.bitcast_convert_type(jnp.int32(0x5F3759DF) - (i >> 1), jnp.float32)
    y = y * (1.5 - 0.5 * d * y * y)
    y = y * (1.5 - 0.5 * d * y * y)
    return y


def _gelu2(v):
    z = v * _SQRT1_2
    a = jnp.abs(z)
    t = _rcp(1.0 + _P * a)
    poly = t * (_A1 + t * (_A2 + t * _A3))
    e = jnp.exp(-(a * a))
    er = 1.0 - poly * e
    sgn = lax.bitcast_convert_type(z, jnp.int32) & jnp.int32(-2147483648)
    er = lax.bitcast_convert_type(
        lax.bitcast_convert_type(er, jnp.int32) | sgn, jnp.float32)
    return v * (1.0 + er)


@functools.lru_cache(maxsize=None)
def _build_sc(N, E, D):
    NC, NT = 2, 16
    NW = NC * NT
    NP = ((N + NT * 640 - 1) // (NT * 640)) * (NT * 640)
    SLICE = NP // NT
    EPT = E // NW
    NB = EPT // _B
    NSB = N // _B
    SPW = (NSB + NW - 1) // NW
    NG = E // (8 * _B)
    NI = (NG + NT - 1) // NT
    assert E % (NW * _B) == 0 and N % _B == 0 and D == 128

    mesh = plsc.VectorSubcoreMesh(core_axis_name="c", subcore_axis_name="s")

    @functools.partial(
        pl.kernel,
        out_type=jax.ShapeDtypeStruct((2, NP, D), jnp.float32),
        mesh=mesh,
        compiler_params=pltpu.CompilerParams(needs_layout_passes=False),
        scratch_types=[
            pltpu.VMEM_SHARED((NP, D), jnp.float32),
            pltpu.VMEM_SHARED((NP,), jnp.float32),
            pltpu.VMEM_SHARED((NP,), jnp.float32),
            pltpu.VMEM_SHARED((NP,), jnp.int32),
            pltpu.VMEM((NP,), jnp.int32),
            pltpu.VMEM((_B, D), jnp.float32),
            pltpu.VMEM((_B, D), jnp.float32),
            pltpu.VMEM((_B, D), jnp.float32),
            pltpu.VMEM((_B,), jnp.int32),
            pltpu.VMEM((_B,), jnp.int32),
            pltpu.VMEM((_B,), jnp.int32),
            pltpu.VMEM((_B,), jnp.int32),
            pltpu.VMEM((_B,), jnp.float32),
            pltpu.VMEM((_B,), jnp.float32),
            pltpu.VMEM((_B + 16,), jnp.float32),
            pltpu.VMEM((_B + 16,), jnp.float32),
            pltpu.VMEM((_B,), jnp.float32),
            pltpu.VMEM((SLICE,), jnp.float32),
            pltpu.VMEM((SLICE,), jnp.float32),
            pltpu.VMEM((SLICE,), jnp.int32),
            pltpu.VMEM((8, _B), jnp.int32),
            pltpu.VMEM((8, _B), jnp.int32),
            pltpu.SemaphoreType.DMA,
            pltpu.SemaphoreType.DMA,
            pltpu.SemaphoreType.DMA,
            pltpu.SemaphoreType.DMA,
        ],
    )
    def sc_kernel(x_hbm, row_hbm, col_hbm, row2_hbm, col2_hbm,
                  ea_hbm, ew_hbm, acc_out,
                  acc_s, degs_s, degd_s, pk_s, pk_t,
                  xe_a, xe_b, msg_t, row_a, row_b, col_a, col_b,
                  ew_a, ew_b, nrm_a, nrm_b, ones_t,
                  zrow_t, deg_t, pkrow_t, rowc_a, colc_a,
                  semA, semBa, semBb, semC):
        c = lax.axis_index("c")
        s = lax.axis_index("s")
        w = c * NT + s
        nbase0 = s * SLICE

        def _z16(i, _):
            for j in range(D // 16):
                msg_t[i, pl.ds(j * 16, 16)] = jnp.zeros((16,), jnp.float32)
            return _
        lax.fori_loop(0, _B, _z16, None)

        def _z16b(i, _):
            zrow_t[pl.ds(i * 16, 16)] = jnp.zeros((16,), jnp.float32)
            return _
        lax.fori_loop(0, SLICE // 16, _z16b, None)
        for g in range(_B // 16):
            ones_t[pl.ds(g * 16, 16)] = jnp.ones((16,), jnp.float32)

        zdescs = []
        for k in range(SLICE // _B):
            zdescs.append(pltpu.async_copy(
                msg_t, acc_s.at[pl.ds(nbase0 + k * _B, _B)], semA))
        zdescs.append(pltpu.async_copy(
            zrow_t, degs_s.at[pl.ds(nbase0, SLICE)], semA))
        zdescs.append(pltpu.async_copy(
            zrow_t, degd_s.at[pl.ds(nbase0, SLICE)], semA))
        for dsc in zdescs:
            dsc.wait()
        plsc.subcore_barrier()

        def _hist(i, _):
            g = i * NT + s

            @pl.when(g < NG)
            def _():
                pltpu.sync_copy(row2_hbm.at[pl.ds(g * 8, 8)], rowc_a)
                pltpu.sync_copy(col2_hbm.at[pl.ds(g * 8, 8)], colc_a)
                descs = []
                for j in range(8):
                    descs.append(pltpu.async_copy(
                        ones_t, degs_s.at[rowc_a.at[j]], semA, add=True))
                    descs.append(pltpu.async_copy(
                        ones_t, degd_s.at[colc_a.at[j]], semA, add=True))
                for dsc in descs:
                    dsc.wait()
            return _
        lax.fori_loop(0, NI, _hist, None)
        plsc.subcore_barrier()

        pltpu.sync_copy(degs_s.at[pl.ds(nbase0, SLICE)], deg_t)
        pltpu.sync_copy(degd_s.at[pl.ds(nbase0, SLICE)], zrow_t)

        def _inv(g, _):
            ds16 = _rsqrt(deg_t[pl.ds(g * 16, 16)] + 1.0)
            dd16 = _rsqrt(zrow_t[pl.ds(g * 16, 16)] + 1.0)
            bs = lax.bitcast_convert_type(ds16, jnp.int32) + jnp.int32(0x8000)
            bd = lax.bitcast_convert_type(dd16, jnp.int32) + jnp.int32(0x8000)
            pkrow_t[pl.ds(g * 16, 16)] = (bs & jnp.int32(-65536)) | (bd >> 16)
            return _
        lax.fori_loop(0, SLICE // 16, _inv, None)
        pltpu.sync_copy(pkrow_t, pk_s.at[pl.ds(nbase0, SLICE)])
        plsc.subcore_barrier()

        pltpu.sync_copy(pk_s, pk_t)

        def _mk_body(xe, nrm):
            def _body(h, _):
                for el in range(2):
                    e = h * 2 + el
                    coef = nrm[pl.ds(e, 16)][0]
                    for j in range(D // 16):
                        v = xe[e, pl.ds(j * 16, 16)]
                        msg_t[e, pl.ds(j * 16, 16)] = _gelu2(v) * coef
                return _
            return _body

        body_a = _mk_body(xe_a, nrm_a)
        body_b = _mk_body(xe_b, nrm_b)

        def _norm(rowX, colX, ewX, nrmX):
            for g in range(_B // 16):
                rv = rowX[pl.ds(g * 16, 16)]
                cv = colX[pl.ds(g * 16, 16)]
                pr = plsc.load_gather(pk_t, [rv])
                pc = plsc.load_gather(pk_t, [cv])
                dsv = lax.bitcast_convert_type(pr & jnp.int32(-65536),
                                               jnp.float32)
                ddv = lax.bitcast_convert_type(pc << 16, jnp.float32)
                wv = ewX[pl.ds(g * 16, 16)]
                nrmX[pl.ds(g * 16, 16)] = dsv * ddv * wv * 0.5

        eb0 = w * EPT

        def _issue(base, rowX, colX, ewX, xeX, semBX):
            pltpu.async_copy(row_hbm.at[pl.ds(base, _B)], rowX, semA)
            pltpu.async_copy(col_hbm.at[pl.ds(base, _B)], colX, semA)
            pltpu.async_copy(ew_hbm.at[pl.ds(base, _B)], ewX, semA)
            pltpu.async_copy(ea_hbm.at[pl.ds(base, _B)], xeX, semBX)

        def _half(k, first, slotX, slotY):
            rowX, colX, ewX, nrmX, xeX, semBX, bodyX = slotX
            rowY, colY, ewY, nrmY, xeY, semBY, _unused = slotY
            base = eb0 + k * _B
            pltpu.make_async_copy(row_hbm.at[pl.ds(base, _B)], rowX,
                                  semA).wait()
            pltpu.make_async_copy(col_hbm.at[pl.ds(base, _B)], colX,
                                  semA).wait()
            pltpu.make_async_copy(ew_hbm.at[pl.ds(base, _B)], ewX,
                                  semA).wait()
            pltpu.make_async_copy(ea_hbm.at[pl.ds(base, _B)], xeX,
                                  semBX).wait()
            gd = pltpu.async_copy(x_hbm.at[rowX], xeX, semBX, add=True)
            _norm(rowX, colX, ewX, nrmX)
            if not first:
                pltpu.make_async_copy(msg_t, acc_s.at[colY], semC).wait()
            if isinstance(k, int):
                if k + 1 < NB:
                    _issue(base + _B, rowY, colY, ewY, xeY, semBY)
            else:
                @pl.when(k + 1 < NB)
                def _():
                    _issue(base + _B, rowY, colY, ewY, xeY, semBY)
            gd.wait()
            lax.fori_loop(0, _B // 2, bodyX, None)
            pltpu.async_copy(msg_t, acc_s.at[colX], semC, add=True)

        slot_a = (row_a, col_a, ew_a, nrm_a, xe_a, semBa, body_a)
        slot_b = (row_b, col_b, ew_b, nrm_b, xe_b, semBb, body_b)

        _issue(eb0, row_a, col_a, ew_a, xe_a, semBa)
        _half(0, True, slot_a, slot_b)

        def _two(i, _):
            k1 = 1 + 2 * i
            _half(k1, False, slot_b, slot_a)
            _half(k1 + 1, False, slot_a, slot_b)
            return _
        lax.fori_loop(0, (NB - 1) // 2, _two, None)
        pltpu.make_async_copy(msg_t, acc_s.at[col_a], semC).wait()

        lane = lax.iota(jnp.int32, 16)
        sb0 = w * SPW
        slots5 = [(xe_a, semBa, col_a, nrm_a, body_a),
                  (xe_b, semBb, col_b, nrm_b, body_b)]

        @pl.when(sb0 < NSB)
        def _():
            pltpu.async_copy(x_hbm.at[pl.ds(sb0 * _B, _B)], xe_a, semBa)

        for b in range(SPW):
            sb = sb0 + b
            xeX, semBX, colX, nrmX, bodyX = slots5[b % 2]
            xeY, semBY, _cy, _ny, _by = slots5[(b + 1) % 2]

            @pl.when(sb < NSB)
            def _():
                nbase = sb * _B
                pltpu.make_async_copy(x_hbm.at[pl.ds(nbase, _B)], xeX,
                                      semBX).wait()
                if b + 1 < SPW:
                    @pl.when(sb + 1 < NSB)
                    def _():
                        pltpu.async_copy(x_hbm.at[pl.ds(nbase + _B, _B)],
                                         xeY, semBY)
                for g in range(_B // 16):
                    off = nbase + g * 16
                    colX[pl.ds(g * 16, 16)] = off + lane
                    pk16 = pk_t[pl.ds(off, 16)]
                    dsv = lax.bitcast_convert_type(pk16 & jnp.int32(-65536),
                                                   jnp.float32)
                    ddv = lax.bitcast_convert_type(pk16 << 16, jnp.float32)
                    nrmX[pl.ds(g * 16, 16)] = dsv * ddv * 0.5
                lax.fori_loop(0, _B // 2, bodyX, None)
                pltpu.sync_copy(msg_t, acc_s.at[colX], add=True)

        plsc.subcore_barrier()

        pltpu.sync_copy(acc_s.at[pl.ds(nbase0, SLICE)],
                        acc_out.at[c, pl.ds(nbase0, SLICE)])

    return sc_kernel


@functools.lru_cache(maxsize=None)
def _build_tc(N, NP, D):
    R = 400
    assert N % R == 0

    def body(acc_ref, w_ref, b_ref, o_ref):
        pre = acc_ref[0] + acc_ref[1]
        o_ref[...] = lax.dot_general(
            pre, w_ref[...], (((1,), (1,)), ((), ())),
            preferred_element_type=jnp.float32) + b_ref[...]

    return pl.pallas_call(
        body,
        grid=(N // R,),
        in_specs=[
            pl.BlockSpec((2, R, D), lambda i: (0, i, 0)),
            pl.BlockSpec((D, D), lambda i: (0, 0)),
            pl.BlockSpec((1, D), lambda i: (0, 0)),
        ],
        out_specs=pl.BlockSpec((R, D), lambda i: (i, 0)),
        out_shape=jax.ShapeDtypeStruct((N, D), jnp.float32),
    )


def kernel(x, edge_index, edge_attr, edge_weight, W, b):
    N, D = x.shape
    E = edge_attr.shape[0]
    row = edge_index[0]
    col = edge_index[1]
    row2 = row.reshape(E // _B, _B)
    col2 = col.reshape(E // _B, _B)
    ew = edge_weight.reshape(E)
    acc = _build_sc(N, E, D)(x, row, col, row2, col2, edge_attr, ew)
    out = _build_tc(N, acc.shape[1], D)(acc, W, b.reshape(1, D))
    return out


# speedup vs baseline: 1.0738x; 1.0738x over previous
import functools

import jax
import jax.numpy as jnp
from jax import lax
from jax.experimental import pallas as pl
from jax.experimental.pallas import tpu as pltpu
from jax.experimental.pallas import tpu_sc as plsc

_B = 80

_SQRT1_2 = 0.7071067811865476
_P = 0.47047
_A1, _A2, _A3 = 0.3480242, -0.0958798, 0.7478556


def _rcp(d):
    i = lax.bitcast_convert_type(d, jnp.int32)
    r = lax.bitcast_convert_type(jnp.int32(0x7EF311C3) - i, jnp.float32)
    r = r * (2.0 - d * r)
    return r


def _rsqrt(d):
    i = lax.bitcast_convert_type(d, jnp.int32)
    y = lax.bitcast_convert_type(jnp.int32(0x5F3759DF) - (i >> 1), jnp.float32)
    y = y * (1.5 - 0.5 * d * y * y)
    y = y * (1.5 - 0.5 * d * y * y)
    return y


def _gelu2(v):
    z = v * _SQRT1_2
    a = jnp.abs(z)
    t = _rcp(1.0 + _P * a)
    poly = t * (_A1 + t * (_A2 + t * _A3))
    e = jnp.exp(-(a * a))
    er = 1.0 - poly * e
    sgn = lax.bitcast_convert_type(z, jnp.int32) & jnp.int32(-2147483648)
    er = lax.bitcast_convert_type(
        lax.bitcast_convert_type(er, jnp.int32) | sgn, jnp.float32)
    return v * (1.0 + er)


@functools.lru_cache(maxsize=None)
def _build_sc(N, E, D):
    NC, NT = 2, 16
    NW = NC * NT
    NP = ((N + NT * 640 - 1) // (NT * 640)) * (NT * 640)
    SLICE = NP // NT
    EPT = E // NW
    NB = EPT // _B
    NSB = N // _B
    SPW = (NSB + NW - 1) // NW
    NG = E // (8 * _B)
    NI = (NG + NT - 1) // NT
    assert E % (NW * _B) == 0 and N % _B == 0 and D == 128

    mesh = plsc.VectorSubcoreMesh(core_axis_name="c", subcore_axis_name="s")

    @functools.partial(
        pl.kernel,
        out_type=jax.ShapeDtypeStruct((2, NP, D), jnp.float32),
        mesh=mesh,
        compiler_params=pltpu.CompilerParams(needs_layout_passes=False),
        scratch_types=[
            pltpu.VMEM_SHARED((NP, D), jnp.float32),
            pltpu.VMEM_SHARED((NP,), jnp.float32),
            pltpu.VMEM_SHARED((NP,), jnp.float32),
            pltpu.VMEM_SHARED((NP,), jnp.int32),
            pltpu.VMEM((NP,), jnp.int32),
            pltpu.VMEM((_B, D), jnp.float32),
            pltpu.VMEM((_B, D), jnp.float32),
            pltpu.VMEM((_B, D), jnp.float32),
            pltpu.VMEM((_B,), jnp.int32),
            pltpu.VMEM((_B,), jnp.int32),
            pltpu.VMEM((_B,), jnp.int32),
            pltpu.VMEM((_B,), jnp.int32),
            pltpu.VMEM((_B,), jnp.float32),
            pltpu.VMEM((_B,), jnp.float32),
            pltpu.VMEM((_B + 16,), jnp.float32),
            pltpu.VMEM((_B + 16,), jnp.float32),
            pltpu.VMEM((_B,), jnp.float32),
            pltpu.VMEM((SLICE,), jnp.float32),
            pltpu.VMEM((SLICE,), jnp.float32),
            pltpu.VMEM((SLICE,), jnp.int32),
            pltpu.VMEM((8, _B), jnp.int32),
            pltpu.VMEM((8, _B), jnp.int32),
            pltpu.SemaphoreType.DMA,
            pltpu.SemaphoreType.DMA,
            pltpu.SemaphoreType.DMA,
            pltpu.SemaphoreType.DMA,
        ],
    )
    def sc_kernel(x_hbm, row_hbm, col_hbm, row2_hbm, col2_hbm,
                  ea_hbm, ew_hbm, acc_out,
                  acc_s, degs_s, degd_s, pk_s, pk_t,
                  xe_a, xe_b, msg_t, row_a, row_b, col_a, col_b,
                  ew_a, ew_b, nrm_a, nrm_b, ones_t,
                  zrow_t, deg_t, pkrow_t, rowc_a, colc_a,
                  semA, semBa, semBb, semC):
        c = lax.axis_index("c")
        s = lax.axis_index("s")
        w = c * NT + s
        nbase0 = s * SLICE

        def _z16(i, _):
            for j in range(D // 16):
                msg_t[i, pl.ds(j * 16, 16)] = jnp.zeros((16,), jnp.float32)
            return _
        lax.fori_loop(0, _B, _z16, None)

        def _z16b(i, _):
            zrow_t[pl.ds(i * 16, 16)] = jnp.zeros((16,), jnp.float32)
            return _
        lax.fori_loop(0, SLICE // 16, _z16b, None)
        for g in range(_B // 16):
            ones_t[pl.ds(g * 16, 16)] = jnp.ones((16,), jnp.float32)

        zdescs = []
        for k in range(SLICE // _B):
            zdescs.append(pltpu.async_copy(
                msg_t, acc_s.at[pl.ds(nbase0 + k * _B, _B)], semA))
        zdescs.append(pltpu.async_copy(
            zrow_t, degs_s.at[pl.ds(nbase0, SLICE)], semA))
        zdescs.append(pltpu.async_copy(
            zrow_t, degd_s.at[pl.ds(nbase0, SLICE)], semA))
        for dsc in zdescs:
            dsc.wait()
        plsc.subcore_barrier()

        def _hist(i, _):
            g = i * NT + s

            @pl.when(g < NG)
            def _():
                pltpu.sync_copy(row2_hbm.at[pl.ds(g * 8, 8)], rowc_a)
                pltpu.sync_copy(col2_hbm.at[pl.ds(g * 8, 8)], colc_a)
                descs = []
                for j in range(8):
                    descs.append(pltpu.async_copy(
                        ones_t, degs_s.at[rowc_a.at[j]], semA, add=True))
                    descs.append(pltpu.async_copy(
                        ones_t, degd_s.at[colc_a.at[j]], semA, add=True))
                for dsc in descs:
                    dsc.wait()
            return _
        lax.fori_loop(0, NI, _hist, None)
        plsc.subcore_barrier()

        pltpu.sync_copy(degs_s.at[pl.ds(nbase0, SLICE)], deg_t)
        pltpu.sync_copy(degd_s.at[pl.ds(nbase0, SLICE)], zrow_t)

        def _inv(g, _):
            ds16 = _rsqrt(deg_t[pl.ds(g * 16, 16)] + 1.0)
            dd16 = _rsqrt(zrow_t[pl.ds(g * 16, 16)] + 1.0)
            bs = lax.bitcast_convert_type(ds16, jnp.int32) + jnp.int32(0x8000)
            bd = lax.bitcast_convert_type(dd16, jnp.int32) + jnp.int32(0x8000)
            pkrow_t[pl.ds(g * 16, 16)] = (bs & jnp.int32(-65536)) | (bd >> 16)
            return _
        lax.fori_loop(0, SLICE // 16, _inv, None)
        pltpu.sync_copy(pkrow_t, pk_s.at[pl.ds(nbase0, SLICE)])
        plsc.subcore_barrier()

        pltpu.sync_copy(pk_s, pk_t)

        def _mk_body(xe, nrm):
            def _body(e, _):
                coef = nrm[pl.ds(e, 16)][0]
                for j in range(D // 16):
                    v = xe[e, pl.ds(j * 16, 16)]
                    msg_t[e, pl.ds(j * 16, 16)] = _gelu2(v) * coef
                return _
            return _body

        body_a = _mk_body(xe_a, nrm_a)
        body_b = _mk_body(xe_b, nrm_b)

        def _norm(rowX, colX, ewX, nrmX):
            for g in range(_B // 16):
                rv = rowX[pl.ds(g * 16, 16)]
                cv = colX[pl.ds(g * 16, 16)]
                pr = plsc.load_gather(pk_t, [rv])
                pc = plsc.load_gather(pk_t, [cv])
                dsv = lax.bitcast_convert_type(pr & jnp.int32(-65536),
                                               jnp.float32)
                ddv = lax.bitcast_convert_type(pc << 16, jnp.float32)
                wv = ewX[pl.ds(g * 16, 16)]
                nrmX[pl.ds(g * 16, 16)] = dsv * ddv * wv * 0.5

        eb0 = w * EPT

        def _issue(base, rowX, colX, ewX, xeX, semBX):
            pltpu.async_copy(row_hbm.at[pl.ds(base, _B)], rowX, semA)
            pltpu.async_copy(col_hbm.at[pl.ds(base, _B)], colX, semA)
            pltpu.async_copy(ew_hbm.at[pl.ds(base, _B)], ewX, semA)
            pltpu.async_copy(ea_hbm.at[pl.ds(base, _B)], xeX, semBX)

        def _half(k, first, slotX, slotY):
            rowX, colX, ewX, nrmX, xeX, semBX, bodyX = slotX
            rowY, colY, ewY, nrmY, xeY, semBY, _unused = slotY
            base = eb0 + k * _B
            pltpu.make_async_copy(row_hbm.at[pl.ds(base, _B)], rowX,
                                  semA).wait()
            pltpu.make_async_copy(col_hbm.at[pl.ds(base, _B)], colX,
                                  semA).wait()
            pltpu.make_async_copy(ew_hbm.at[pl.ds(base, _B)], ewX,
                                  semA).wait()
            pltpu.make_async_copy(ea_hbm.at[pl.ds(base, _B)], xeX,
                                  semBX).wait()
            gd = pltpu.async_copy(x_hbm.at[rowX], xeX, semBX, add=True)
            _norm(rowX, colX, ewX, nrmX)
            if not first:
                pltpu.make_async_copy(msg_t, acc_s.at[colY], semC).wait()
            if isinstance(k, int):
                if k + 1 < NB:
                    _issue(base + _B, rowY, colY, ewY, xeY, semBY)
            else:
                @pl.when(k + 1 < NB)
                def _():
                    _issue(base + _B, rowY, colY, ewY, xeY, semBY)
            gd.wait()
            lax.fori_loop(0, _B, bodyX, None)
            pltpu.async_copy(msg_t, acc_s.at[colX], semC, add=True)

        slot_a = (row_a, col_a, ew_a, nrm_a, xe_a, semBa, body_a)
        slot_b = (row_b, col_b, ew_b, nrm_b, xe_b, semBb, body_b)

        _issue(eb0, row_a, col_a, ew_a, xe_a, semBa)
        _half(0, True, slot_a, slot_b)

        def _two(i, _):
            k1 = 1 + 2 * i
            _half(k1, False, slot_b, slot_a)
            _half(k1 + 1, False, slot_a, slot_b)
            return _
        lax.fori_loop(0, (NB - 1) // 2, _two, None)
        pltpu.make_async_copy(msg_t, acc_s.at[col_a], semC).wait()

        lane = lax.iota(jnp.int32, 16)
        sb0 = w * SPW
        slots5 = [(xe_a, semBa, col_a, nrm_a, body_a),
                  (xe_b, semBb, col_b, nrm_b, body_b)]

        @pl.when(sb0 < NSB)
        def _():
            pltpu.async_copy(x_hbm.at[pl.ds(sb0 * _B, _B)], xe_a, semBa)

        for b in range(SPW):
            sb = sb0 + b
            xeX, semBX, colX, nrmX, bodyX = slots5[b % 2]
            xeY, semBY, _cy, _ny, _by = slots5[(b + 1) % 2]

            @pl.when(sb < NSB)
            def _():
                nbase = sb * _B
                pltpu.make_async_copy(x_hbm.at[pl.ds(nbase, _B)], xeX,
                                      semBX).wait()
                if b + 1 < SPW:
                    @pl.when(sb + 1 < NSB)
                    def _():
                        pltpu.async_copy(x_hbm.at[pl.ds(nbase + _B, _B)],
                                         xeY, semBY)
                for g in range(_B // 16):
                    off = nbase + g * 16
                    colX[pl.ds(g * 16, 16)] = off + lane
                    pk16 = pk_t[pl.ds(off, 16)]
                    dsv = lax.bitcast_convert_type(pk16 & jnp.int32(-65536),
                                                   jnp.float32)
                    ddv = lax.bitcast_convert_type(pk16 << 16, jnp.float32)
                    nrmX[pl.ds(g * 16, 16)] = dsv * ddv * 0.5
                lax.fori_loop(0, _B, bodyX, None)
                pltpu.sync_copy(msg_t, acc_s.at[colX], add=True)

        plsc.subcore_barrier()

        pltpu.sync_copy(acc_s.at[pl.ds(nbase0, SLICE)],
                        acc_out.at[c, pl.ds(nbase0, SLICE)])

    return sc_kernel


@functools.lru_cache(maxsize=None)
def _build_tc(N, NP, D):
    R = 400
    assert N % R == 0

    def body(acc_ref, w_ref, b_ref, o_ref):
        pre = acc_ref[0] + acc_ref[1]
        o_ref[...] = lax.dot_general(
            pre, w_ref[...], (((1,), (1,)), ((), ())),
            preferred_element_type=jnp.float32) + b_ref[...]

    return pl.pallas_call(
        body,
        grid=(N // R,),
        in_specs=[
            pl.BlockSpec((2, R, D), lambda i: (0, i, 0)),
            pl.BlockSpec((D, D), lambda i: (0, 0)),
            pl.BlockSpec((1, D), lambda i: (0, 0)),
        ],
        out_specs=pl.BlockSpec((R, D), lambda i: (i, 0)),
        out_shape=jax.ShapeDtypeStruct((N, D), jnp.float32),
    )


def kernel(x, edge_index, edge_attr, edge_weight, W, b):
    N, D = x.shape
    E = edge_attr.shape[0]
    row = edge_index[0]
    col = edge_index[1]
    row2 = row.reshape(E // _B, _B)
    col2 = col.reshape(E // _B, _B)
    ew = edge_weight.reshape(E)
    acc = _build_sc(N, E, D)(x, row, col, row2, col2, edge_attr, ew)
    out = _build_tc(N, acc.shape[1], D)(acc, W, b.reshape(1, D))
    return out
